# baseline (speedup 1.0000x reference)
import functools

import jax
import jax.numpy as jnp
import numpy as np
from jax.experimental import pallas as pl
from jax.experimental.pallas import tpu as pltpu

_B = 16384
_A = 128
_R = 4
_K = _A * _R

_BS = 512


def _body(q_ref, w_ref, p_ref, g_ref, h_ref, o_ref):
    q2 = q_ref[:]
    w = w_ref[:]
    wfull = jnp.dot(w, p_ref[:], preferred_element_type=jnp.float32,
                    precision=jax.lax.Precision.HIGHEST)
    t = q2 * wfull
    prod = jnp.dot(t, g_ref[:], preferred_element_type=jnp.float32,
                   precision=jax.lax.Precision.HIGHEST)
    a = jnp.argmax(prod, axis=1).astype(jnp.int32)
    iota = jax.lax.broadcasted_iota(jnp.int32, (_BS, _K), 1)
    ohfull = (iota >> 2) == a[:, None]
    t3 = jnp.where(ohfull, q2, 0.0)
    o_ref[:] = jnp.dot(t3, h_ref[:], preferred_element_type=jnp.float32,
                       precision=jax.lax.Precision.HIGHEST)


@jax.jit
def kernel(q, w):
    bq = q.shape[0] // _A
    q2 = q.reshape(bq, _K)
    k_idx = np.arange(_K)
    p_mat = jnp.asarray((np.arange(_R)[:, None] == (k_idx[None, :] % _R)),
                        dtype=jnp.float32)
    g_mat = jnp.asarray(((k_idx[:, None] // _R) == np.arange(_A)[None, :]),
                        dtype=jnp.float32)
    h_mat = jnp.asarray(((k_idx[:, None] % _R) == np.arange(_R)[None, :]),
                        dtype=jnp.float32)

    grid = (bq // _BS,)
    out = pl.pallas_call(
        _body,
        grid=grid,
        in_specs=[
            pl.BlockSpec((_BS, _K), lambda i: (i, 0)),
            pl.BlockSpec((_BS, _R), lambda i: (i, 0)),
            pl.BlockSpec((_R, _K), lambda i: (0, 0)),
            pl.BlockSpec((_K, _A), lambda i: (0, 0)),
            pl.BlockSpec((_K, _R), lambda i: (0, 0)),
        ],
        out_specs=pl.BlockSpec((_BS, _R), lambda i: (i, 0)),
        out_shape=jax.ShapeDtypeStruct((bq, _R), jnp.float32),
        compiler_params=pltpu.CompilerParams(
            dimension_semantics=("arbitrary",),
        ),
    )(q2, w, p_mat, g_mat, h_mat)
    return out


# speedup vs baseline: 1.0048x; 1.0048x over previous
import functools

import jax
import jax.numpy as jnp
import numpy as np
from jax.experimental import pallas as pl
from jax.experimental.pallas import tpu as pltpu

_B = 16384
_A = 128
_R = 4
_K = _A * _R

_BS = 512


def _body(q_ref, w_ref, p_ref, g_ref, h_ref, o_ref):
    qb = q_ref[:]
    q2 = qb.reshape(_BS, _K)
    w = w_ref[:]
    wfull = jnp.dot(w, p_ref[:], preferred_element_type=jnp.float32,
                    precision=jax.lax.Precision.HIGHEST)
    t = q2 * wfull
    prod = jnp.dot(t, g_ref[:], preferred_element_type=jnp.float32,
                   precision=jax.lax.Precision.HIGHEST)
    a = jnp.argmax(prod, axis=1).astype(jnp.int32)
    iota = jax.lax.broadcasted_iota(jnp.int32, (_BS, _K), 1)
    ohfull = (iota >> 2) == a[:, None]
    t3 = jnp.where(ohfull, q2, 0.0)
    o_ref[:] = jnp.dot(t3, h_ref[:], preferred_element_type=jnp.float32,
                       precision=jax.lax.Precision.HIGHEST)


@jax.jit
def kernel(q, w):
    bq = q.shape[0] // _A
    qf = q.reshape(bq * 4, _A)
    k_idx = np.arange(_K)
    p_mat = jnp.asarray((np.arange(_R)[:, None] == (k_idx[None, :] % _R)),
                        dtype=jnp.float32)
    g_mat = jnp.asarray(((k_idx[:, None] // _R) == np.arange(_A)[None, :]),
                        dtype=jnp.float32)
    h_mat = jnp.asarray(((k_idx[:, None] % _R) == np.arange(_R)[None, :]),
                        dtype=jnp.float32)

    grid = (bq // _BS,)
    out = pl.pallas_call(
        _body,
        grid=grid,
        in_specs=[
            pl.BlockSpec((_BS * 4, _A), lambda i: (i, 0)),
            pl.BlockSpec((_BS, _R), lambda i: (i, 0)),
            pl.BlockSpec((_R, _K), lambda i: (0, 0)),
            pl.BlockSpec((_K, _A), lambda i: (0, 0)),
            pl.BlockSpec((_K, _R), lambda i: (0, 0)),
        ],
        out_specs=pl.BlockSpec((_BS, _R), lambda i: (i, 0)),
        out_shape=jax.ShapeDtypeStruct((bq, _R), jnp.float32),
        compiler_params=pltpu.CompilerParams(
            dimension_semantics=("arbitrary",),
        ),
    )(qf, w, p_mat, g_mat, h_mat)
    return out


# speedup vs baseline: 55.0445x; 54.7791x over previous
import jax
import jax.numpy as jnp
from jax.experimental import pallas as pl
from jax.experimental.pallas import tpu as pltpu

_A = 128
_R = 4
_K = _A * _R

_BS = 512


def _body(q_ref, w_ref, o_ref):
    qb = q_ref[:]
    q2 = qb.reshape(_BS, _K)
    w = w_ref[:]
    s = [q2[:, r * _A:(r + 1) * _A] for r in range(_R)]
    prod = s[0] * w[:, 0:1] + s[1] * w[:, 1:2] + s[2] * w[:, 2:3] + s[3] * w[:, 3:4]
    a_star = jnp.argmax(prod, axis=1).astype(jnp.int32)
    iota = jax.lax.broadcasted_iota(jnp.int32, (_BS, _A), 1)
    oh = iota == a_star[:, None]
    cols = [jnp.sum(jnp.where(oh, s[r], 0.0), axis=1, keepdims=True)
            for r in range(_R)]
    o_ref[:] = jnp.concatenate(cols, axis=1)


@jax.jit
def kernel(q, w):
    bq = q.shape[0] // _A
    qt2 = q.reshape(bq, _A, _R).transpose(0, 2, 1).reshape(bq * _R, _A)

    grid = (bq // _BS,)
    out = pl.pallas_call(
        _body,
        grid=grid,
        in_specs=[
            pl.BlockSpec((_BS * _R, _A), lambda i: (i, 0)),
            pl.BlockSpec((_BS, _R), lambda i: (i, 0)),
        ],
        out_specs=pl.BlockSpec((_BS, _R), lambda i: (i, 0)),
        out_shape=jax.ShapeDtypeStruct((bq, _R), jnp.float32),
        compiler_params=pltpu.CompilerParams(
            dimension_semantics=("arbitrary",),
        ),
    )(qt2, w)
    return out


# speedup vs baseline: 63.5864x; 1.1552x over previous
import jax
import jax.numpy as jnp
import numpy as np
from jax.experimental import pallas as pl
from jax.experimental.pallas import tpu as pltpu

_A = 128
_R = 4
_K = _A * _R

_BS = 512
_NB = _BS // 128


def _body(q_ref, w_ref, h_ref, o_ref):
    qb = q_ref[:]
    q2 = qb.reshape(_BS, _K)
    s = [q2[:, r * _A:(r + 1) * _A] for r in range(_R)]

    wt = jnp.transpose(w_ref[:])
    wblk = jnp.concatenate([wt[:, _R * bb:_R * (bb + 1)]
                            for bb in range(_NB)], axis=0)

    prod = (s[0] * wblk[:, 0:1] + s[1] * wblk[:, 1:2]
            + s[2] * wblk[:, 2:3] + s[3] * wblk[:, 3:4])
    a_star = jnp.argmax(prod, axis=1).astype(jnp.int32)

    iota = jax.lax.broadcasted_iota(jnp.int32, (_BS, _K), 1)
    oh4 = (iota & (_A - 1)) == a_star[:, None]
    selq = jnp.where(oh4, q2, 0.0)
    moq = jnp.dot(selq, h_ref[:], preferred_element_type=jnp.float32)
    moqw = jnp.concatenate([moq[_A * bb:_A * (bb + 1), :]
                            for bb in range(_NB)], axis=1)
    o_ref[:] = jnp.transpose(moqw)


@jax.jit
def kernel(q, w):
    bq = q.shape[0] // _A
    qt2 = q.reshape(bq, _A, _R).transpose(0, 2, 1).reshape(bq * _R, _A)
    wt2 = w.reshape(bq // _A, _A, _R).transpose(0, 2, 1).reshape(bq * _R // _A, _A)
    h_mat = jnp.asarray(
        (np.arange(_K)[:, None] // _A) == np.arange(_R)[None, :],
        dtype=jnp.float32)

    grid = (bq // _BS,)
    ot = pl.pallas_call(
        _body,
        grid=grid,
        in_specs=[
            pl.BlockSpec((_BS * _R, _A), lambda i: (i, 0)),
            pl.BlockSpec((_R * _NB, _A), lambda i: (i, 0)),
            pl.BlockSpec((_K, _R), lambda i: (0, 0)),
        ],
        out_specs=pl.BlockSpec((_R * _NB, _A), lambda i: (i, 0)),
        out_shape=jax.ShapeDtypeStruct((bq * _R // _A, _A), jnp.float32),
        compiler_params=pltpu.CompilerParams(
            dimension_semantics=("arbitrary",),
        ),
    )(qt2, wt2, h_mat)
    return ot.reshape(bq // _A, _R, _A).transpose(0, 2, 1).reshape(bq, _R)


# speedup vs baseline: 66.7255x; 1.0494x over previous
import jax
import jax.numpy as jnp
from jax.experimental import pallas as pl
from jax.experimental.pallas import tpu as pltpu

_A = 128
_R = 4
_K = _A * _R

_BS = 512
_NB = _BS // 128


def _body(q_ref, w_ref, o_ref):
    qb = q_ref[:]
    q2 = qb.reshape(_BS, _K)
    s = [q2[:, r * _A:(r + 1) * _A] for r in range(_R)]

    wt = jnp.transpose(w_ref[:])
    wblk = jnp.concatenate([wt[:, _R * bb:_R * (bb + 1)]
                            for bb in range(_NB)], axis=0)

    prod = (s[0] * wblk[:, 0:1] + s[1] * wblk[:, 1:2]
            + s[2] * wblk[:, 2:3] + s[3] * wblk[:, 3:4])
    a_star = jnp.argmax(prod, axis=1).astype(jnp.int32)

    iota = jax.lax.broadcasted_iota(jnp.int32, (_BS, _A), 1)
    oh = iota == a_star[:, None]
    moq = jnp.concatenate(
        [jnp.sum(jnp.where(oh, s[r], 0.0), axis=1, keepdims=True)
         for r in range(_R)], axis=1)
    moqw = jnp.concatenate([moq[_A * bb:_A * (bb + 1), :]
                            for bb in range(_NB)], axis=1)
    o_ref[:] = jnp.transpose(moqw)


@jax.jit
def kernel(q, w):
    bq = q.shape[0] // _A
    qt2 = q.reshape(bq, _A, _R).transpose(0, 2, 1).reshape(bq * _R, _A)
    wt2 = w.reshape(bq // _A, _A, _R).transpose(0, 2, 1).reshape(bq * _R // _A, _A)
    grid = (bq // _BS,)
    ot = pl.pallas_call(
        _body,
        grid=grid,
        in_specs=[
            pl.BlockSpec((_BS * _R, _A), lambda i: (i, 0)),
            pl.BlockSpec((_R * _NB, _A), lambda i: (i, 0)),
        ],
        out_specs=pl.BlockSpec((_R * _NB, _A), lambda i: (i, 0)),
        out_shape=jax.ShapeDtypeStruct((bq * _R // _A, _A), jnp.float32),
        compiler_params=pltpu.CompilerParams(
            dimension_semantics=("arbitrary",),
        ),
    )(qt2, wt2)
    return ot.reshape(bq // _A, _R, _A).transpose(0, 2, 1).reshape(bq, _R)


# speedup vs baseline: 73.4039x; 1.1001x over previous
import jax
import jax.numpy as jnp
from jax.experimental import pallas as pl
from jax.experimental.pallas import tpu as pltpu

_A = 128
_R = 4
_K = _A * _R

_BS = 1024
_NB = _BS // 128


def _body(q_ref, w_ref, o_ref):
    qb = q_ref[:]
    q2 = qb.reshape(_BS, _K)
    s = [q2[:, r * _A:(r + 1) * _A] for r in range(_R)]

    wt = jnp.transpose(w_ref[:])
    wblk = jnp.concatenate([wt[:, _R * bb:_R * (bb + 1)]
                            for bb in range(_NB)], axis=0)

    prod = (s[0] * wblk[:, 0:1] + s[1] * wblk[:, 1:2]
            + s[2] * wblk[:, 2:3] + s[3] * wblk[:, 3:4])
    a_star = jnp.argmax(prod, axis=1).astype(jnp.int32)

    iota = jax.lax.broadcasted_iota(jnp.int32, (_BS, _A), 1)
    oh = iota == a_star[:, None]
    moq = jnp.concatenate(
        [jnp.sum(jnp.where(oh, s[r], 0.0), axis=1, keepdims=True)
         for r in range(_R)], axis=1)
    moqw = jnp.concatenate([moq[_A * bb:_A * (bb + 1), :]
                            for bb in range(_NB)], axis=1)
    o_ref[:] = jnp.transpose(moqw)


@jax.jit
def kernel(q, w):
    bq = q.shape[0] // _A
    qt2 = q.reshape(bq, _A, _R).transpose(0, 2, 1).reshape(bq * _R, _A)
    wt2 = w.reshape(bq // _A, _A, _R).transpose(0, 2, 1).reshape(bq * _R // _A, _A)
    grid = (bq // _BS,)
    ot = pl.pallas_call(
        _body,
        grid=grid,
        in_specs=[
            pl.BlockSpec((_BS * _R, _A), lambda i: (i, 0)),
            pl.BlockSpec((_R * _NB, _A), lambda i: (i, 0)),
        ],
        out_specs=pl.BlockSpec((_R * _NB, _A), lambda i: (i, 0)),
        out_shape=jax.ShapeDtypeStruct((bq * _R // _A, _A), jnp.float32),
        compiler_params=pltpu.CompilerParams(
            dimension_semantics=("arbitrary",),
        ),
    )(qt2, wt2)
    return ot.reshape(bq // _A, _R, _A).transpose(0, 2, 1).reshape(bq, _R)


# speedup vs baseline: 74.3646x; 1.0131x over previous
import jax
import jax.numpy as jnp
from jax.experimental import pallas as pl
from jax.experimental.pallas import tpu as pltpu

_A = 128
_R = 4
_K = _A * _R

_BS = 2048
_NB = _BS // 128


def _body(q_ref, w_ref, o_ref):
    qb = q_ref[:]
    q2 = qb.reshape(_BS, _K)
    s = [q2[:, r * _A:(r + 1) * _A] for r in range(_R)]

    wt = jnp.transpose(w_ref[:])
    wblk = jnp.concatenate([wt[:, _R * bb:_R * (bb + 1)]
                            for bb in range(_NB)], axis=0)

    prod = (s[0] * wblk[:, 0:1] + s[1] * wblk[:, 1:2]
            + s[2] * wblk[:, 2:3] + s[3] * wblk[:, 3:4])
    a_star = jnp.argmax(prod, axis=1).astype(jnp.int32)

    iota = jax.lax.broadcasted_iota(jnp.int32, (_BS, _A), 1)
    oh = iota == a_star[:, None]
    moq = jnp.concatenate(
        [jnp.sum(jnp.where(oh, s[r], 0.0), axis=1, keepdims=True)
         for r in range(_R)], axis=1)
    moqw = jnp.concatenate([moq[_A * bb:_A * (bb + 1), :]
                            for bb in range(_NB)], axis=1)
    o_ref[:] = jnp.transpose(moqw)


@jax.jit
def kernel(q, w):
    bq = q.shape[0] // _A
    qt2 = q.reshape(bq, _A, _R).transpose(0, 2, 1).reshape(bq * _R, _A)
    wt2 = w.reshape(bq // _A, _A, _R).transpose(0, 2, 1).reshape(bq * _R // _A, _A)
    grid = (bq // _BS,)
    ot = pl.pallas_call(
        _body,
        grid=grid,
        in_specs=[
            pl.BlockSpec((_BS * _R, _A), lambda i: (i, 0)),
            pl.BlockSpec((_R * _NB, _A), lambda i: (i, 0)),
        ],
        out_specs=pl.BlockSpec((_R * _NB, _A), lambda i: (i, 0)),
        out_shape=jax.ShapeDtypeStruct((bq * _R // _A, _A), jnp.float32),
        compiler_params=pltpu.CompilerParams(
            dimension_semantics=("arbitrary",),
        ),
    )(qt2, wt2)
    return ot.reshape(bq // _A, _R, _A).transpose(0, 2, 1).reshape(bq, _R)
